# baseline (speedup 1.0000x reference)
import functools

import jax
import jax.numpy as jnp
from jax import lax
from jax.experimental import pallas as pl
from jax.experimental.pallas import tpu as pltpu
from jax.experimental.pallas import tpu_sc as plsc

B = 4096
F = 26
D = 64
BF = B * F
NC, NS = 2, 16
NW = NC * NS
PER_W = BF // NW
CHUNK = 128
NCH = PER_W // CHUNK
LANES = 16
FIELD_SIZE = 100000


def _body(x_hbm, w_hbm, out_hbm, idx_v, rows_v, sem):
    wid = lax.axis_index("s") * NC + lax.axis_index("c")
    base = wid * PER_W

    def chunk(j, carry):
        start = base + j * CHUNK
        pltpu.sync_copy(x_hbm.at[pl.ds(start, CHUNK)], idx_v)
        for t in range(CHUNK // LANES):
            pos = start + t * LANES + lax.iota(jnp.int32, LANES)
            fld = lax.rem(pos, F)
            sl = pl.ds(t * LANES, LANES)
            idx_v[sl] = idx_v[sl] + fld * FIELD_SIZE
        pltpu.async_copy(w_hbm.at[idx_v], rows_v, sem).wait()
        pltpu.sync_copy(rows_v, out_hbm.at[pl.ds(start, CHUNK)])
        return carry

    lax.fori_loop(0, NCH, chunk, 0)


def kernel(x, weight):
    mesh = plsc.VectorSubcoreMesh(
        core_axis_name="c", subcore_axis_name="s",
        num_cores=NC, num_subcores=NS,
    )
    lookup = pl.kernel(
        _body,
        out_type=jax.ShapeDtypeStruct((BF, D), jnp.float32),
        mesh=mesh,
        scratch_types=[
            pltpu.VMEM((CHUNK,), jnp.int32),
            pltpu.VMEM((CHUNK, D), jnp.float32),
            pltpu.SemaphoreType.DMA,
        ],
        compiler_params=pltpu.CompilerParams(use_tc_tiling_on_sc=False),
    )
    out = lookup(x.reshape(BF), weight)
    return out.reshape(B, F * D)


# speedup vs baseline: 1.0169x; 1.0169x over previous
import jax
import jax.numpy as jnp
from jax import lax
from jax.experimental import pallas as pl
from jax.experimental.pallas import tpu as pltpu
from jax.experimental.pallas import tpu_sc as plsc

B = 4096
F = 26
D = 64
BF = B * F
NC, NS = 2, 16
NW = NC * NS
PER_W = BF // NW
CHUNK = 832
NCH = PER_W // CHUNK
NBUF = 2
LANES = 16
FIELD_SIZE = 100000


def _body(x_hbm, w_hbm, out_hbm, idx_v, buf0, buf1, gs0, gs1, cs0, cs1):
    wid = lax.axis_index("s") * NC + lax.axis_index("c")
    base = wid * PER_W
    pltpu.sync_copy(x_hbm.at[pl.ds(base, PER_W)], idx_v)

    def off(t, carry):
        pos = base + t * LANES + lax.iota(jnp.int32, LANES)
        sl = pl.ds(t * LANES, LANES)
        idx_v[sl] = idx_v[sl] + lax.rem(pos, F) * FIELD_SIZE
        return carry

    lax.fori_loop(0, PER_W // LANES, off, 0)

    bufs = (buf0, buf1)
    gsems = (gs0, gs1)
    csems = (cs0, cs1)
    gathers = [
        pltpu.async_copy(
            w_hbm.at[idx_v.at[pl.ds(b * CHUNK, CHUNK)]], bufs[b], gsems[b])
        for b in range(NBUF)
    ]
    copies = [None] * NBUF
    for j in range(NCH):
        b = j % NBUF
        gathers[b].wait()
        copies[b] = pltpu.async_copy(
            bufs[b], out_hbm.at[pl.ds(base + j * CHUNK, CHUNK)], csems[b])
        nj = j + NBUF
        if nj < NCH:
            copies[b].wait()
            gathers[b] = pltpu.async_copy(
                w_hbm.at[idx_v.at[pl.ds(nj * CHUNK, CHUNK)]], bufs[b], gsems[b])
    for j in range(max(0, NCH - NBUF), NCH):
        copies[j % NBUF].wait()


def kernel(x, weight):
    mesh = plsc.VectorSubcoreMesh(
        core_axis_name="c", subcore_axis_name="s",
        num_cores=NC, num_subcores=NS,
    )
    lookup = pl.kernel(
        _body,
        out_type=jax.ShapeDtypeStruct((BF, D), jnp.float32),
        mesh=mesh,
        scratch_types=[
            pltpu.VMEM((PER_W,), jnp.int32),
            pltpu.VMEM((CHUNK, D), jnp.float32),
            pltpu.VMEM((CHUNK, D), jnp.float32),
            pltpu.SemaphoreType.DMA,
            pltpu.SemaphoreType.DMA,
            pltpu.SemaphoreType.DMA,
            pltpu.SemaphoreType.DMA,
        ],
        compiler_params=pltpu.CompilerParams(use_tc_tiling_on_sc=False),
    )
    out = lookup(x.reshape(BF), weight)
    return out.reshape(B, F * D)
